# baseline (speedup 1.0000x reference)
import functools

import jax
import jax.numpy as jnp
from jax import lax
from jax.experimental import pallas as pl
from jax.experimental.pallas import tpu as pltpu
from jax.experimental.pallas import tpu_sc as plsc

LANES = 16


@functools.lru_cache(maxsize=None)
def _make_kernel(B, D):
    info = plsc.get_sparse_core_info()
    NC, NS = info.num_cores, info.num_subcores
    NW = NC * NS
    assert B % NW == 0 and D % LANES == 0
    b_per_w = B // NW
    mesh = plsc.VectorSubcoreMesh(core_axis_name="c", subcore_axis_name="s")

    @functools.partial(
        pl.kernel,
        mesh=mesh,
        compiler_params=pltpu.CompilerParams(use_tc_tiling_on_sc=False),
        out_type=jax.ShapeDtypeStruct((B, D), jnp.float32),
        scratch_types=[
            pltpu.VMEM((b_per_w,), jnp.int32),
            pltpu.VMEM((b_per_w,), jnp.int32),
            pltpu.VMEM((b_per_w, D), jnp.float32),
            pltpu.VMEM((b_per_w, D), jnp.float32),
            pltpu.SemaphoreType.DMA,
            pltpu.SemaphoreType.DMA,
        ],
    )
    def gmf(uids, sids, utab, stab, out, uidx, sidx, urows, srows, sem_u, sem_s):
        wid = lax.axis_index("s") * NC + lax.axis_index("c")
        base = wid * b_per_w
        pltpu.sync_copy(uids.at[pl.ds(base, b_per_w)], uidx)
        pltpu.sync_copy(sids.at[pl.ds(base, b_per_w)], sidx)
        cu = pltpu.async_copy(utab.at[uidx], urows, sem_u)
        cs = pltpu.async_copy(stab.at[sidx], srows, sem_s)
        cu.wait()
        cs.wait()

        def body(i, carry):
            for j in range(D // LANES):
                sl = pl.ds(j * LANES, LANES)
                urows[i, sl] = urows[i, sl] * srows[i, sl]
            return carry

        lax.fori_loop(0, b_per_w, body, 0)
        pltpu.sync_copy(urows, out.at[pl.ds(base, b_per_w)])

    return gmf


def kernel(users_ids, services_ids, user_table, service_table):
    B, = users_ids.shape
    D = user_table.shape[1]
    gmf = _make_kernel(B, D)
    return gmf(
        users_ids.astype(jnp.int32),
        services_ids.astype(jnp.int32),
        user_table,
        service_table,
    )


# speedup vs baseline: 1.4986x; 1.4986x over previous
import functools

import jax
import jax.numpy as jnp
from jax import lax
from jax.experimental import pallas as pl
from jax.experimental.pallas import tpu as pltpu
from jax.experimental.pallas import tpu_sc as plsc

LANES = 16
ROWS = 256


@functools.lru_cache(maxsize=None)
def _make_kernel(B, D):
    info = plsc.get_sparse_core_info()
    NC, NS = info.num_cores, info.num_subcores
    NW = NC * NS
    assert B % NW == 0 and D % LANES == 0
    b_per_w = B // NW
    n_pass = b_per_w // ROWS
    mesh = plsc.VectorSubcoreMesh(core_axis_name="c", subcore_axis_name="s")

    @functools.partial(
        pl.kernel,
        mesh=mesh,
        out_type=jax.ShapeDtypeStruct((B, D), jnp.float32),
        scratch_types=[
            pltpu.VMEM((b_per_w,), jnp.int32),
            pltpu.VMEM((b_per_w,), jnp.int32),
            pltpu.VMEM((ROWS, D), jnp.float32),
            pltpu.VMEM((ROWS, D), jnp.float32),
            pltpu.SemaphoreType.DMA,
            pltpu.SemaphoreType.DMA,
        ],
    )
    def gmf(uids, sids, utab, stab, out, uidx, sidx,
            urows, srows, sem_u, sem_s):
        wid = lax.axis_index("s") * NC + lax.axis_index("c")
        base = wid * b_per_w
        pltpu.sync_copy(uids.at[pl.ds(base, b_per_w)], uidx)
        pltpu.sync_copy(sids.at[pl.ds(base, b_per_w)], sidx)

        for h in range(n_pass):
            lo = h * ROWS

            def issue(g, carry):
                gbase = g * LANES
                uvec = uidx[pl.ds(lo + gbase, LANES)]
                svec = sidx[pl.ds(lo + gbase, LANES)]
                for j in range(LANES):
                    pltpu.make_async_copy(
                        utab.at[uvec[j]], urows.at[gbase + j], sem_u).start()
                    pltpu.make_async_copy(
                        stab.at[svec[j]], srows.at[gbase + j], sem_s).start()
                return carry

            lax.fori_loop(0, ROWS // LANES, issue, 0)
            pltpu.make_async_copy(utab.at[pl.ds(0, ROWS)], urows, sem_u).wait()
            pltpu.make_async_copy(stab.at[pl.ds(0, ROWS)], srows, sem_s).wait()

            def body(i, carry):
                for j in range(D // LANES):
                    sl = pl.ds(j * LANES, LANES)
                    urows[i, sl] = urows[i, sl] * srows[i, sl]
                return carry

            lax.fori_loop(0, ROWS, body, 0)
            pltpu.sync_copy(urows, out.at[pl.ds(base + lo, ROWS)])

    return gmf


def kernel(users_ids, services_ids, user_table, service_table):
    B, = users_ids.shape
    D = user_table.shape[1]
    gmf = _make_kernel(B, D)
    return gmf(
        users_ids.astype(jnp.int32),
        services_ids.astype(jnp.int32),
        user_table,
        service_table,
    )


# speedup vs baseline: 2.2123x; 1.4762x over previous
import functools

import jax
import jax.numpy as jnp
from jax import lax
from jax.experimental import pallas as pl
from jax.experimental.pallas import tpu as pltpu
from jax.experimental.pallas import tpu_sc as plsc

LANES = 16
CHUNK = 32
SLAB = 256


@functools.lru_cache(maxsize=None)
def _make_kernel(B, D):
    info = plsc.get_sparse_core_info()
    NC, NS = info.num_cores, info.num_subcores
    NW = NC * NS
    assert B % NW == 0 and D % LANES == 0
    b_per_w = B // NW
    assert b_per_w % SLAB == 0 and SLAB % CHUNK == 0
    mesh = plsc.VectorSubcoreMesh(core_axis_name="c", subcore_axis_name="s")

    @functools.partial(
        pl.kernel,
        mesh=mesh,
        out_type=jax.ShapeDtypeStruct((B, D), jnp.float32),
        scratch_types=[
            pltpu.VMEM((b_per_w,), jnp.int32),
            pltpu.VMEM((b_per_w,), jnp.int32),
            pltpu.VMEM((CHUNK, 8, D), jnp.float32),
            pltpu.VMEM((CHUNK, 8, D), jnp.float32),
            pltpu.VMEM((SLAB, D), jnp.float32),
            pltpu.SemaphoreType.DMA,
            pltpu.SemaphoreType.DMA,
        ],
    )
    def gmf(uids, sids, utab3, stab3, out, uidx, sidx,
            ublk, sblk, prod, sem_u, sem_s):
        wid = lax.axis_index("s") * NC + lax.axis_index("c")
        base = wid * b_per_w
        pltpu.sync_copy(uids.at[pl.ds(base, b_per_w)], uidx)
        pltpu.sync_copy(sids.at[pl.ds(base, b_per_w)], sidx)

        def chunk_body(c, carry):
            lo = c * CHUNK
            uvecs = [uidx[pl.ds(lo + g * LANES, LANES)]
                     for g in range(CHUNK // LANES)]
            svecs = [sidx[pl.ds(lo + g * LANES, LANES)]
                     for g in range(CHUNK // LANES)]
            for g in range(CHUNK // LANES):
                for j in range(LANES):
                    i = g * LANES + j
                    pltpu.make_async_copy(
                        utab3.at[uvecs[g][j] >> 3], ublk.at[i], sem_u).start()
                    pltpu.make_async_copy(
                        stab3.at[svecs[g][j] >> 3], sblk.at[i], sem_s).start()
            pltpu.make_async_copy(utab3.at[pl.ds(0, CHUNK)], ublk, sem_u).wait()
            pltpu.make_async_copy(stab3.at[pl.ds(0, CHUNK)], sblk, sem_s).wait()
            pb = (c % (SLAB // CHUNK)) * CHUNK
            for g in range(CHUNK // LANES):
                for j in range(LANES):
                    i = g * LANES + j
                    ur = uvecs[g][j] & 7
                    sr = svecs[g][j] & 7
                    for k in range(D // LANES):
                        sl = pl.ds(k * LANES, LANES)
                        prod[pb + i, sl] = ublk[i, ur, sl] * sblk[i, sr, sl]
            return carry

        n_per_slab = SLAB // CHUNK

        for h in range(b_per_w // SLAB):
            lax.fori_loop(h * n_per_slab, (h + 1) * n_per_slab, chunk_body, 0)
            pltpu.sync_copy(prod, out.at[pl.ds(base + h * SLAB, SLAB)])

    return gmf


def kernel(users_ids, services_ids, user_table, service_table):
    B, = users_ids.shape
    V, D = user_table.shape
    gmf = _make_kernel(B, D)
    return gmf(
        users_ids.astype(jnp.int32),
        services_ids.astype(jnp.int32),
        user_table.reshape(V // 8, 8, D),
        service_table.reshape(V // 8, 8, D),
    )
